# baseline (speedup 1.0000x reference)
import functools

import jax
import jax.numpy as jnp
from jax import lax
from jax.experimental import pallas as pl
from jax.experimental.pallas import tpu as pltpu
from jax.experimental.pallas import tpu_sc as plsc

B = 16384
L = 20
V = 100000
CV = 1000
SV = 1000
D = 128
TD = 32
CD = 32

NC = 2
NS = 16
NW = NC * NS
BPW = B // NW
CH = 4
IDXC = CH * L
TCHUNK = BPW // CH
CROWS = 64

_mesh = plsc.VectorSubcoreMesh(core_axis_name="c", subcore_axis_name="s")


@functools.partial(
    pl.kernel,
    out_type=jax.ShapeDtypeStruct((B, D), jnp.float32),
    mesh=_mesh,
    scratch_types=[
        pltpu.VMEM((TCHUNK, IDXC), jnp.int32),
        pltpu.VMEM((IDXC, D), jnp.float32),
        pltpu.VMEM((IDXC, D), jnp.float32),
        pltpu.VMEM((IDXC, D), jnp.float32),
        pltpu.VMEM((IDXC, D), jnp.float32),
        pltpu.VMEM((BPW, D), jnp.float32),
        pltpu.SemaphoreType.DMA,
        pltpu.SemaphoreType.DMA,
        pltpu.SemaphoreType.DMA,
        pltpu.SemaphoreType.DMA,
    ],
)
def _sc_gather(title_r, wword, tout,
               tidx, ring0, ring1, ring2, ring3, stage,
               sem0, sem1, sem2, sem3):
    wid = lax.axis_index("s") * NC + lax.axis_index("c")
    base = wid * BPW

    pltpu.sync_copy(title_r.at[pl.ds(wid * TCHUNK, TCHUNK)], tidx)

    rings = (ring0, ring1, ring2, ring3)
    sems = (sem0, sem1, sem2, sem3)

    def fire(ch, b):
        pltpu.async_copy(wword.at[tidx.at[ch]], rings[b], sems[b])

    def wait(ch, b):
        pltpu.make_async_copy(wword.at[tidx.at[ch]], rings[b], sems[b]).wait()

    def reduce(ch, b):
        ring = rings[b]

        @pl.loop(0, CH)
        def _(t):
            row = ch * CH + t
            tb = t * L
            for j in range(D // 16):
                col = pl.ds(j * 16, 16)
                a = [ring[tb + 2 * p, col] + ring[tb + 2 * p + 1, col]
                     for p in range(L // 2)]
                while len(a) > 1:
                    a = [a[i] + a[i + 1] for i in range(0, len(a) - 1, 2)] \
                        + ([a[-1]] if len(a) % 2 else [])
                stage[row, col] = a[0]

    fire(0, 0)
    fire(1, 1)
    fire(2, 2)

    @pl.loop(0, TCHUNK - 4, step=4)
    def _(cch):
        for b4 in range(4):
            ch = cch + b4
            fire(ch + 3, (b4 + 3) % 4)
            wait(ch, b4)
            reduce(ch, b4)

    ch0 = TCHUNK - 4
    fire(TCHUNK - 1, (TCHUNK - 1) % 4)
    for b4 in range(4):
        ch = ch0 + b4
        wait(ch, ch % 4)
        reduce(ch, ch % 4)

    pltpu.sync_copy(stage, tout.at[pl.ds(base, BPW)])


@functools.partial(
    pl.kernel,
    out_type=[
        jax.ShapeDtypeStruct((B, D), jnp.float32),
        jax.ShapeDtypeStruct((B, D), jnp.float32),
    ],
    mesh=_mesh,
    scratch_types=[
        pltpu.VMEM((CROWS, D), jnp.float32),
        pltpu.VMEM((CROWS, D), jnp.float32),
        pltpu.VMEM((BPW // CROWS, CROWS), jnp.int32),
        pltpu.VMEM((BPW // CROWS, CROWS), jnp.int32),
        pltpu.SemaphoreType.DMA,
        pltpu.SemaphoreType.DMA,
    ],
)
def _sc_catsub(cat_r, sub_r, wcat, wsub, cout, sout,
               crow0, crow1, cidx, sidx, csem0, csem1):
    wid = lax.axis_index("s") * NC + lax.axis_index("c")
    base = wid * BPW
    pltpu.sync_copy(cat_r.at[pl.ds(wid * (BPW // CROWS), BPW // CROWS)], cidx)
    pltpu.sync_copy(sub_r.at[pl.ds(wid * (BPW // CROWS), BPW // CROWS)], sidx)

    nk = BPW // CROWS
    cat_jobs = ([(wcat, cidx, cout, k) for k in range(nk)]
                + [(wsub, sidx, sout, k) for k in range(nk)])
    crows = (crow0, crow1)
    csems = (csem0, csem1)

    def cat_fire(j, b):
        tbl, idx, _, k = cat_jobs[j]
        pltpu.async_copy(tbl.at[idx.at[k]], crows[b], csems[b])

    def cat_wait(j, b):
        tbl, idx, _, k = cat_jobs[j]
        pltpu.make_async_copy(tbl.at[idx.at[k]], crows[b], csems[b]).wait()

    cat_fire(0, 0)
    cat_fire(1, 1)
    for j in range(2 * nk):
        b = j % 2
        cat_wait(j, b)
        if j + 2 < 2 * nk:
            cat_fire(j + 2, b)
        _, _, out, k = cat_jobs[j]
        pltpu.sync_copy(crows[b], out.at[pl.ds(base + k * CROWS, CROWS)])


BLK = 2048


def _tc_body(ts_ref, cv_ref, sv_ref, w1t_ref, b1_ref,
             wf1t_ref, wf2t_ref, wf3t_ref, bf_ref, o_ref):
    t = jnp.dot(ts_ref[...], w1t_ref[...], preferred_element_type=jnp.float32)
    t = jnp.maximum(t + b1_ref[...], 0.0)
    y = (jnp.dot(t, wf1t_ref[...], preferred_element_type=jnp.float32)
         + jnp.dot(cv_ref[...][:, :CD], wf2t_ref[...],
                   preferred_element_type=jnp.float32)
         + jnp.dot(sv_ref[...][:, :CD], wf3t_ref[...],
                   preferred_element_type=jnp.float32)
         + bf_ref[...])
    o_ref[...] = jnp.maximum(y, 0.0)


_tc_dense = pl.pallas_call(
    _tc_body,
    grid=(B // BLK,),
    in_specs=[
        pl.BlockSpec((BLK, D), lambda i: (i, 0)),
        pl.BlockSpec((BLK, D), lambda i: (i, 0)),
        pl.BlockSpec((BLK, D), lambda i: (i, 0)),
        pl.BlockSpec((D, TD), lambda i: (0, 0)),
        pl.BlockSpec((1, TD), lambda i: (0, 0)),
        pl.BlockSpec((TD, D), lambda i: (0, 0)),
        pl.BlockSpec((CD, D), lambda i: (0, 0)),
        pl.BlockSpec((CD, D), lambda i: (0, 0)),
        pl.BlockSpec((1, D), lambda i: (0, 0)),
    ],
    out_specs=pl.BlockSpec((BLK, D), lambda i: (i, 0)),
    out_shape=jax.ShapeDtypeStruct((B, D), jnp.float32),
)


def kernel(title, category, subcategory, W_word, W_title_reduce,
           b_title_reduce, W_cat, W_subcat, W_final, b_final):
    cat_r = category.astype(jnp.int32).reshape(B // CROWS, CROWS)
    sub_r = subcategory.astype(jnp.int32).reshape(B // CROWS, CROWS)
    wcat_p = jnp.pad(W_cat, ((0, 0), (0, D - CD)))
    wsub_p = jnp.pad(W_subcat, ((0, 0), (0, D - CD)))
    catv, subv = _sc_catsub(cat_r, sub_r, wcat_p, wsub_p)

    title_r = title.astype(jnp.int32).reshape(NW * TCHUNK, IDXC)
    tsum = _sc_gather(title_r, W_word)

    w1t = W_title_reduce.T * (1.0 / L)
    wf1t = W_final[:, :TD].T
    wf2t = W_final[:, TD:TD + CD].T
    wf3t = W_final[:, TD + CD:].T
    return _tc_dense(tsum, catv, subv, w1t,
                     b_title_reduce.reshape(1, TD), wf1t, wf2t, wf3t,
                     b_final.reshape(1, D))


# speedup vs baseline: 1.0576x; 1.0576x over previous
import functools

import jax
import jax.numpy as jnp
from jax import lax
from jax.experimental import pallas as pl
from jax.experimental.pallas import tpu as pltpu
from jax.experimental.pallas import tpu_sc as plsc

B = 16384
L = 20
V = 100000
CV = 1000
SV = 1000
D = 128
TD = 32
CD = 32

NC = 2
NS = 16
NW = NC * NS
BPW = B // NW
CH = 4
IDXC = CH * L
TCHUNK = BPW // CH
CROWS = 64

_mesh = plsc.VectorSubcoreMesh(core_axis_name="c", subcore_axis_name="s")


@functools.partial(
    pl.kernel,
    out_type=jax.ShapeDtypeStruct((B, D), jnp.float32),
    mesh=_mesh,
    scratch_types=[
        pltpu.VMEM((TCHUNK, IDXC), jnp.int32),
        pltpu.VMEM((IDXC, D), jnp.float32),
        pltpu.VMEM((IDXC, D), jnp.float32),
        pltpu.VMEM((IDXC, D), jnp.float32),
        pltpu.VMEM((IDXC, D), jnp.float32),
        pltpu.VMEM((BPW, D), jnp.float32),
        pltpu.SemaphoreType.DMA,
        pltpu.SemaphoreType.DMA,
        pltpu.SemaphoreType.DMA,
        pltpu.SemaphoreType.DMA,
    ],
)
def _sc_gather(title_r, wword, tout,
               tidx, ring0, ring1, ring2, ring3, stage,
               sem0, sem1, sem2, sem3):
    wid = lax.axis_index("s") * NC + lax.axis_index("c")
    base = wid * BPW

    pltpu.sync_copy(title_r.at[pl.ds(wid * TCHUNK, TCHUNK)], tidx)

    rings = (ring0, ring1, ring2, ring3)
    sems = (sem0, sem1, sem2, sem3)

    def fire(ch, b):
        pltpu.async_copy(wword.at[tidx.at[ch]], rings[b], sems[b])

    def wait(ch, b):
        pltpu.make_async_copy(wword.at[tidx.at[ch]], rings[b], sems[b]).wait()

    def reduce(ch, b):
        ring = rings[b]

        @pl.loop(0, CH)
        def _(t):
            row = ch * CH + t
            tb = t * L
            for j in range(D // 16):
                col = pl.ds(j * 16, 16)
                a = [ring[tb + 2 * p, col] + ring[tb + 2 * p + 1, col]
                     for p in range(L // 2)]
                while len(a) > 1:
                    a = [a[i] + a[i + 1] for i in range(0, len(a) - 1, 2)] \
                        + ([a[-1]] if len(a) % 2 else [])
                stage[row, col] = a[0]

    fire(0, 0)
    fire(1, 1)
    fire(2, 2)

    @pl.loop(0, TCHUNK - 4, step=4)
    def _(cch):
        for b4 in range(4):
            ch = cch + b4
            fire(ch + 3, (b4 + 3) % 4)
            wait(ch, b4)
            reduce(ch, b4)

    ch0 = TCHUNK - 4
    fire(TCHUNK - 1, (TCHUNK - 1) % 4)
    for b4 in range(4):
        ch = ch0 + b4
        wait(ch, ch % 4)
        reduce(ch, ch % 4)

    pltpu.sync_copy(stage, tout.at[pl.ds(base, BPW)])


BLK = 2048
CVP = 1024


def _tc_body(ts_ref, cat_ref, sub_ref, w1t_ref, b1_ref,
             wcat_ref, wsub_ref, wf1t_ref, wf2t_ref, wf3t_ref, bf_ref,
             o_ref):
    t = jnp.dot(ts_ref[...], w1t_ref[...], preferred_element_type=jnp.float32)
    t = jnp.maximum(t + b1_ref[...], 0.0)
    iota = lax.broadcasted_iota(jnp.int32, (BLK, CVP), 1)
    ohc = jnp.where(iota == cat_ref[...], 1.0, 0.0)
    ohs = jnp.where(iota == sub_ref[...], 1.0, 0.0)
    cv = jnp.dot(ohc, wcat_ref[...], preferred_element_type=jnp.float32)
    sv = jnp.dot(ohs, wsub_ref[...], preferred_element_type=jnp.float32)
    y = (jnp.dot(t, wf1t_ref[...], preferred_element_type=jnp.float32)
         + jnp.dot(cv, wf2t_ref[...], preferred_element_type=jnp.float32)
         + jnp.dot(sv, wf3t_ref[...], preferred_element_type=jnp.float32)
         + bf_ref[...])
    o_ref[...] = jnp.maximum(y, 0.0)


_tc_dense = pl.pallas_call(
    _tc_body,
    grid=(B // BLK,),
    in_specs=[
        pl.BlockSpec((BLK, D), lambda i: (i, 0)),
        pl.BlockSpec((BLK, 1), lambda i: (i, 0)),
        pl.BlockSpec((BLK, 1), lambda i: (i, 0)),
        pl.BlockSpec((D, TD), lambda i: (0, 0)),
        pl.BlockSpec((1, TD), lambda i: (0, 0)),
        pl.BlockSpec((CVP, CD), lambda i: (0, 0)),
        pl.BlockSpec((CVP, CD), lambda i: (0, 0)),
        pl.BlockSpec((TD, D), lambda i: (0, 0)),
        pl.BlockSpec((CD, D), lambda i: (0, 0)),
        pl.BlockSpec((CD, D), lambda i: (0, 0)),
        pl.BlockSpec((1, D), lambda i: (0, 0)),
    ],
    out_specs=pl.BlockSpec((BLK, D), lambda i: (i, 0)),
    out_shape=jax.ShapeDtypeStruct((B, D), jnp.float32),
)


def kernel(title, category, subcategory, W_word, W_title_reduce,
           b_title_reduce, W_cat, W_subcat, W_final, b_final):
    cat_c = category.astype(jnp.int32).reshape(B, 1)
    sub_c = subcategory.astype(jnp.int32).reshape(B, 1)
    wcat_p = jnp.pad(W_cat, ((0, CVP - CV), (0, 0)))
    wsub_p = jnp.pad(W_subcat, ((0, CVP - SV), (0, 0)))

    title_r = title.astype(jnp.int32).reshape(NW * TCHUNK, IDXC)
    tsum = _sc_gather(title_r, W_word)

    w1t = W_title_reduce.T * (1.0 / L)
    wf1t = W_final[:, :TD].T
    wf2t = W_final[:, TD:TD + CD].T
    wf3t = W_final[:, TD + CD:].T
    return _tc_dense(tsum, cat_c, sub_c, w1t,
                     b_title_reduce.reshape(1, TD), wcat_p, wsub_p,
                     wf1t, wf2t, wf3t, b_final.reshape(1, D))


# speedup vs baseline: 1.0620x; 1.0041x over previous
import functools

import jax
import jax.numpy as jnp
from jax import lax
from jax.experimental import pallas as pl
from jax.experimental.pallas import tpu as pltpu
from jax.experimental.pallas import tpu_sc as plsc

B = 16384
L = 20
V = 100000
CV = 1000
SV = 1000
D = 128
TD = 32
CD = 32

NC = 2
NS = 16
NW = NC * NS
CH = 4
IDXC = CH * L
NHALF = 2
BH = B // NHALF

_mesh = plsc.VectorSubcoreMesh(core_axis_name="c", subcore_axis_name="s")


def _make_sc_gather(nb):
    bpw = nb // NW
    tchunk = bpw // CH

    @functools.partial(
        pl.kernel,
        out_type=jax.ShapeDtypeStruct((nb, D), jnp.float32),
        mesh=_mesh,
        scratch_types=[
            pltpu.VMEM((tchunk, IDXC), jnp.int32),
            pltpu.VMEM((IDXC, D), jnp.float32),
            pltpu.VMEM((IDXC, D), jnp.float32),
            pltpu.VMEM((IDXC, D), jnp.float32),
            pltpu.VMEM((IDXC, D), jnp.float32),
            pltpu.VMEM((bpw, D), jnp.float32),
            pltpu.SemaphoreType.DMA,
            pltpu.SemaphoreType.DMA,
            pltpu.SemaphoreType.DMA,
            pltpu.SemaphoreType.DMA,
        ],
    )
    def _sc_gather(title_r, wword, tout,
                   tidx, ring0, ring1, ring2, ring3, stage,
                   sem0, sem1, sem2, sem3):
        wid = lax.axis_index("s") * NC + lax.axis_index("c")
        base = wid * bpw

        pltpu.sync_copy(title_r.at[pl.ds(wid * tchunk, tchunk)], tidx)

        rings = (ring0, ring1, ring2, ring3)
        sems = (sem0, sem1, sem2, sem3)

        def fire(ch, b):
            pltpu.async_copy(wword.at[tidx.at[ch]], rings[b], sems[b])

        def wait(ch, b):
            pltpu.make_async_copy(wword.at[tidx.at[ch]], rings[b],
                                  sems[b]).wait()

        def reduce(ch, b):
            ring = rings[b]

            @pl.loop(0, CH)
            def _(t):
                row = ch * CH + t
                tb = t * L
                for j in range(D // 16):
                    col = pl.ds(j * 16, 16)
                    a = [ring[tb + 2 * p, col] + ring[tb + 2 * p + 1, col]
                         for p in range(L // 2)]
                    while len(a) > 1:
                        a = ([a[i] + a[i + 1]
                              for i in range(0, len(a) - 1, 2)]
                             + ([a[-1]] if len(a) % 2 else []))
                    stage[row, col] = a[0]

        fire(0, 0)
        fire(1, 1)
        fire(2, 2)

        @pl.loop(0, tchunk - 4, step=4)
        def _(cch):
            for b4 in range(4):
                ch = cch + b4
                fire(ch + 3, (b4 + 3) % 4)
                wait(ch, b4)
                reduce(ch, b4)

        ch0 = tchunk - 4
        fire(tchunk - 1, (tchunk - 1) % 4)
        for b4 in range(4):
            ch = ch0 + b4
            wait(ch, ch % 4)
            reduce(ch, ch % 4)

        pltpu.sync_copy(stage, tout.at[pl.ds(base, bpw)])

    return _sc_gather


_sc_gather_half = _make_sc_gather(BH)

BLK = 2048
CVP = 1024


def _tc_body(ts_ref, cat_ref, sub_ref, w1t_ref, b1_ref,
             wcat_ref, wsub_ref, wf1t_ref, wf2t_ref, wf3t_ref, bf_ref,
             o_ref):
    t = jnp.dot(ts_ref[...], w1t_ref[...], preferred_element_type=jnp.float32)
    t = jnp.maximum(t + b1_ref[...], 0.0)
    iota = lax.broadcasted_iota(jnp.int32, (BLK, CVP), 1)
    ohc = jnp.where(iota == cat_ref[...], 1.0, 0.0)
    ohs = jnp.where(iota == sub_ref[...], 1.0, 0.0)
    cv = jnp.dot(ohc, wcat_ref[...], preferred_element_type=jnp.float32)
    sv = jnp.dot(ohs, wsub_ref[...], preferred_element_type=jnp.float32)
    y = (jnp.dot(t, wf1t_ref[...], preferred_element_type=jnp.float32)
         + jnp.dot(cv, wf2t_ref[...], preferred_element_type=jnp.float32)
         + jnp.dot(sv, wf3t_ref[...], preferred_element_type=jnp.float32)
         + bf_ref[...])
    o_ref[...] = jnp.maximum(y, 0.0)


_tc_dense_half = pl.pallas_call(
    _tc_body,
    grid=(BH // BLK,),
    in_specs=[
        pl.BlockSpec((BLK, D), lambda i: (i, 0)),
        pl.BlockSpec((BLK, 1), lambda i: (i, 0)),
        pl.BlockSpec((BLK, 1), lambda i: (i, 0)),
        pl.BlockSpec((D, TD), lambda i: (0, 0)),
        pl.BlockSpec((1, TD), lambda i: (0, 0)),
        pl.BlockSpec((CVP, CD), lambda i: (0, 0)),
        pl.BlockSpec((CVP, CD), lambda i: (0, 0)),
        pl.BlockSpec((TD, D), lambda i: (0, 0)),
        pl.BlockSpec((CD, D), lambda i: (0, 0)),
        pl.BlockSpec((CD, D), lambda i: (0, 0)),
        pl.BlockSpec((1, D), lambda i: (0, 0)),
    ],
    out_specs=pl.BlockSpec((BLK, D), lambda i: (i, 0)),
    out_shape=jax.ShapeDtypeStruct((BH, D), jnp.float32),
)


def kernel(title, category, subcategory, W_word, W_title_reduce,
           b_title_reduce, W_cat, W_subcat, W_final, b_final):
    cat_c = category.astype(jnp.int32).reshape(B, 1)
    sub_c = subcategory.astype(jnp.int32).reshape(B, 1)
    wcat_p = jnp.pad(W_cat, ((0, CVP - CV), (0, 0)))
    wsub_p = jnp.pad(W_subcat, ((0, CVP - SV), (0, 0)))

    title_i = title.astype(jnp.int32)

    w1t = W_title_reduce.T * (1.0 / L)
    wf1t = W_final[:, :TD].T
    wf2t = W_final[:, TD:TD + CD].T
    wf3t = W_final[:, TD + CD:].T
    b1 = b_title_reduce.reshape(1, TD)
    bf = b_final.reshape(1, D)

    tsums = []
    for h in range(NHALF):
        trh = title_i[h * BH:(h + 1) * BH].reshape(BH * L // IDXC, IDXC)
        tsums.append(_sc_gather_half(trh, W_word))
    outs = []
    for h in range(NHALF):
        outs.append(_tc_dense_half(
            tsums[h],
            cat_c[h * BH:(h + 1) * BH], sub_c[h * BH:(h + 1) * BH],
            w1t, b1, wcat_p, wsub_p, wf1t, wf2t, wf3t, bf))
    return jnp.concatenate(outs, axis=0)
